# baseline (speedup 1.0000x reference)
import functools

import jax
import jax.numpy as jnp
from jax import lax
from jax.experimental import pallas as pl
from jax.experimental.pallas import tpu as pltpu
from jax.experimental.pallas import tpu_sc as plsc

N = 10000
H = 128
ANN = 2
E = 320000
ELIG = 5000
A_SIZE = 32
FC = 64

NC = 2
NS = 16
L = 16
NW = NC * NS

CHUNK = 128
CHUNKS_PER_TILE = 79
TOTAL_CHUNKS_PAD = NW * CHUNKS_PER_TILE
E_PAD = TOTAL_CHUNKS_PAD * CHUNK
ACC_ROWS = 10240
STRIPE = ACC_ROWS // NS
LAST_ROWS = N - (NS - 1) * STRIPE

BN = 2000


def _stage_a_body(h0_ref, annp_ref, wmT_ref, bm_ref, whT_ref, bh_ref,
                  h_ref, hm_ref, gh_ref):
    h = h0_ref[...] + annp_ref[...]
    h_ref[...] = h
    hm_ref[...] = jnp.dot(h, wmT_ref[...],
                          preferred_element_type=jnp.float32) + bm_ref[...]
    gh_ref[...] = jnp.dot(h, whT_ref[...],
                          preferred_element_type=jnp.float32) + bh_ref[...]


def _stage_a(h0, annp, wmT, bm, whT, bh):
    grid = (N // BN,)
    row_spec = pl.BlockSpec((BN, H), lambda i: (i, 0))
    full = lambda shape: pl.BlockSpec(shape, lambda i: (0,) * len(shape))
    return pl.pallas_call(
        _stage_a_body,
        grid=grid,
        in_specs=[row_spec, row_spec, full((H, H)), full((1, H)),
                  full((H, 3 * H)), full((1, 3 * H))],
        out_specs=[row_spec, row_spec,
                   pl.BlockSpec((BN, 3 * H), lambda i: (i, 0))],
        out_shape=[jax.ShapeDtypeStruct((N, H), jnp.float32),
                   jax.ShapeDtypeStruct((N, H), jnp.float32),
                   jax.ShapeDtypeStruct((N, 3 * H), jnp.float32)],
    )(h0, annp, wmT, bm, whT, bh)


@functools.cache
def _mesh():
    return plsc.VectorSubcoreMesh(core_axis_name="c", subcore_axis_name="s",
                                  num_cores=NC, num_subcores=NS)


@functools.cache
def _build_scatter_kernel():
  @functools.partial(
      pl.kernel,
      out_type=jax.ShapeDtypeStruct((NC, N, H), jnp.float32),
      mesh=_mesh(),
      scratch_types=[
          pltpu.VMEM((CHUNKS_PER_TILE, CHUNK), jnp.int32),
          pltpu.VMEM((CHUNKS_PER_TILE, CHUNK), jnp.int32),
          pltpu.VMEM((CHUNK, H), jnp.float32),
          pltpu.VMEM_SHARED((ACC_ROWS, H), jnp.float32),
          pltpu.SemaphoreType.DMA,
      ],
      compiler_params=pltpu.CompilerParams(needs_layout_passes=False),
  )
  def _scatter_kernel(hm_hbm, src_hbm, dst_hbm, zeros_hbm, out_hbm,
                      sidx_v, didx_v, rows_v, acc_sh, sem):
    c = lax.axis_index("c")
    s = lax.axis_index("s")
    wid = s * NC + c

    r0 = s * STRIPE
    pltpu.sync_copy(zeros_hbm.at[pl.ds(r0, STRIPE)],
                    acc_sh.at[pl.ds(r0, STRIPE)])
    plsc.subcore_barrier()

    pltpu.sync_copy(src_hbm.at[wid], sidx_v)
    pltpu.sync_copy(dst_hbm.at[wid], didx_v)

    def body(t, carry):
        pltpu.async_copy(hm_hbm.at[sidx_v.at[t]], rows_v, sem).wait()
        pltpu.sync_copy(rows_v, acc_sh.at[didx_v.at[t]], add=True)
        return carry

    lax.fori_loop(0, CHUNKS_PER_TILE, body, 0)

    plsc.subcore_barrier()
    @pl.when(s < NS - 1)
    def _():
        pltpu.sync_copy(acc_sh.at[pl.ds(r0, STRIPE)],
                        out_hbm.at[c, pl.ds(r0, STRIPE)])

    @pl.when(s == NS - 1)
    def _():
        pltpu.sync_copy(acc_sh.at[pl.ds((NS - 1) * STRIPE, LAST_ROWS)],
                        out_hbm.at[c, pl.ds((NS - 1) * STRIPE, LAST_ROWS)])

  return _scatter_kernel


def _stage_c_body(p0_ref, p1_ref, h_ref, gh_ref, wiT_ref, bi_ref,
                  w1T_ref, b1_ref, w2T_ref, b2_ref, w3_ref, b3_ref,
                  hid_ref, sel_ref):
    incoming = p0_ref[...] + p1_ref[...]
    gi = jnp.dot(incoming, wiT_ref[...],
                 preferred_element_type=jnp.float32) + bi_ref[...]
    gh = gh_ref[...]
    h = h_ref[...]
    r = jax.nn.sigmoid(gi[:, :H] + gh[:, :H])
    z = jax.nn.sigmoid(gi[:, H:2 * H] + gh[:, H:2 * H])
    n = jnp.tanh(gi[:, 2 * H:] + r * gh[:, 2 * H:])
    hid = (1.0 - z) * n + z * h
    hid_ref[...] = hid
    x = jax.nn.relu(jnp.dot(hid, w1T_ref[...],
                            preferred_element_type=jnp.float32) + b1_ref[...])
    x = jax.nn.relu(jnp.dot(x, w2T_ref[...],
                            preferred_element_type=jnp.float32) + b2_ref[...])
    sel_ref[...] = (jnp.sum(x * w3_ref[...], axis=1, keepdims=True)
                    + b3_ref[...])


def _stage_c(p0, p1, h, gh, wiT, bi, w1T, b1, w2T, b2, w3, b3):
    grid = (N // BN,)
    row_spec = pl.BlockSpec((BN, H), lambda i: (i, 0))
    row3_spec = pl.BlockSpec((BN, 3 * H), lambda i: (i, 0))
    full = lambda shape: pl.BlockSpec(shape, lambda i: (0,) * len(shape))
    return pl.pallas_call(
        _stage_c_body,
        grid=grid,
        in_specs=[row_spec, row_spec, row_spec, row3_spec,
                  full((H, 3 * H)), full((1, 3 * H)),
                  full((H, FC)), full((1, FC)),
                  full((FC, FC)), full((1, FC)),
                  full((1, FC)), full((1, 1))],
        out_specs=[row_spec, pl.BlockSpec((BN, 1), lambda i: (i, 0))],
        out_shape=[jax.ShapeDtypeStruct((N, H), jnp.float32),
                   jax.ShapeDtypeStruct((N, 1), jnp.float32)],
    )(p0, p1, h, gh, wiT, bi, w1T, b1, w2T, b2, w3, b3)


_N_FULL = ELIG // L
_REM = ELIG - _N_FULL * L
_EC = 128
_E_CHUNKS = -(-ELIG // _EC)
_ELIG_PAD = _E_CHUNKS * _EC
_BIG = 2147483647


@functools.cache
def _build_argmax_kernel():
  @functools.partial(
      pl.kernel,
      out_type=(jax.ShapeDtypeStruct((L,), jnp.int32),
                jax.ShapeDtypeStruct((L, H), jnp.float32)),
      mesh=_mesh(),
      scratch_types=[
          pltpu.VMEM((_E_CHUNKS, _EC), jnp.int32),
          pltpu.VMEM((_E_CHUNKS, _EC), jnp.float32),
          pltpu.VMEM((L,), jnp.int32),
          pltpu.VMEM((L, H), jnp.float32),
          pltpu.SemaphoreType.DMA,
      ],
      compiler_params=pltpu.CompilerParams(needs_layout_passes=False),
  )
  def _argmax_kernel(sel_hbm, elig_hbm, hid_hbm, idx_out, row_out,
                     elig_v, vals_v, nidx_v, row_v, sem):
    c = lax.axis_index("c")
    s = lax.axis_index("s")

    @pl.when(jnp.logical_and(c == 0, s == 0))
    def _():
        pltpu.sync_copy(elig_hbm, elig_v)
        descs = [
            pltpu.async_copy(sel_hbm.at[elig_v.at[t]], vals_v.at[t], sem)
            for t in range(_E_CHUNKS)
        ]
        for d in descs:
            d.wait()

        neg = jnp.full((L,), -3.0e38, jnp.float32)
        lane = lax.iota(jnp.int32, L)
        per_row = _EC // L

        def body(j, carry):
            best_val, best_pos, best_node = carry
            row = j // per_row
            col = (j % per_row) * L
            vals = vals_v[row, pl.ds(col, L)]
            nodes = elig_v[row, pl.ds(col, L)]
            pos = j * L + lane
            better = vals > best_val
            return (jnp.where(better, vals, best_val),
                    jnp.where(better, pos, best_pos),
                    jnp.where(better, nodes, best_node))

        bv, bp, bn = lax.fori_loop(
            0, _N_FULL, body,
            (neg, jnp.zeros((L,), jnp.int32), jnp.zeros((L,), jnp.int32)))
        tmask = lane < _REM
        row = _N_FULL // per_row
        col = (_N_FULL % per_row) * L
        vals_t = jnp.where(tmask, vals_v[row, pl.ds(col, L)], neg)
        nodes_t = elig_v[row, pl.ds(col, L)]
        pos_t = _N_FULL * L + lane
        better = vals_t > bv
        bv = jnp.where(better, vals_t, bv)
        bp = jnp.where(better, pos_t, bp)
        bn = jnp.where(better, nodes_t, bn)
        m = jnp.max(bv)
        rel = jnp.min(jnp.where(bv == m, bp, _BIG))
        node = jnp.min(jnp.where(bp == rel, bn, _BIG))
        nidx_v[...] = jnp.full((L,), node, jnp.int32)
        pltpu.sync_copy(nidx_v, idx_out)
        pltpu.async_copy(hid_hbm.at[nidx_v], row_v, sem).wait()
        pltpu.sync_copy(row_v, row_out)

  return _argmax_kernel


def _stage_e_body(row_ref, w1T_ref, b1_ref, w2T_ref, b2_ref, w3T_ref, b3_ref,
                  out_ref):
    x = row_ref[...]
    x = jax.nn.relu(jnp.dot(x, w1T_ref[...],
                            preferred_element_type=jnp.float32) + b1_ref[...])
    x = jax.nn.relu(jnp.dot(x, w2T_ref[...],
                            preferred_element_type=jnp.float32) + b2_ref[...])
    out_ref[...] = jnp.dot(x, w3T_ref[...],
                           preferred_element_type=jnp.float32) + b3_ref[...]


def _stage_e(row8, w1T, b1, w2T, b2, w3T, b3):
    return pl.pallas_call(
        _stage_e_body,
        out_shape=jax.ShapeDtypeStruct((8, A_SIZE), jnp.float32),
    )(row8, w1T, b1, w2T, b2, w3T, b3)


def kernel(initial_node_representation, annotations, edge_index, eligibleNodes,
           W_msg, b_msg, gru_wi, gru_wh, gru_bi, gru_bh,
           sel_w1, sel_b1, sel_w2, sel_b2, sel_w3, sel_b3,
           col_w1, col_b1, col_w2, col_b2, col_w3, col_b3):
    annp = jnp.pad(annotations, ((0, 0), (0, H - ANN)))

    h, hm, gh = _stage_a(
        initial_node_representation, annp,
        W_msg.T, b_msg.reshape(1, H),
        gru_wh.T, gru_bh.reshape(1, 3 * H))

    pad = E_PAD - E
    src_p = jnp.concatenate(
        [edge_index[0], jnp.zeros((pad,), jnp.int32)]
    ).reshape(NW, CHUNKS_PER_TILE, CHUNK)
    dst_p = jnp.concatenate(
        [edge_index[1], jnp.full((pad,), N, jnp.int32)]
    ).reshape(NW, CHUNKS_PER_TILE, CHUNK)
    zeros = jnp.zeros((ACC_ROWS, H), jnp.float32)

    partials = _build_scatter_kernel()(hm, src_p, dst_p, zeros)

    hidden, select_out = _stage_c(
        partials[0], partials[1], h, gh,
        gru_wi.T, gru_bi.reshape(1, 3 * H),
        sel_w1.T, sel_b1.reshape(1, FC),
        sel_w2.T, sel_b2.reshape(1, FC),
        sel_w3.reshape(1, FC), sel_b3.reshape(1, 1))

    elig_p = jnp.concatenate(
        [eligibleNodes, jnp.zeros((_ELIG_PAD - ELIG,), jnp.int32)]
    ).reshape(_E_CHUNKS, _EC)
    idx_vec, row = _build_argmax_kernel()(select_out.reshape(N),
                                          elig_p, hidden)
    node_index = idx_vec[0]

    color8 = _stage_e(
        row[:8],
        col_w1.T, col_b1.reshape(1, FC),
        col_w2.T, col_b2.reshape(1, FC),
        col_w3.T, col_b3.reshape(1, A_SIZE))
    color_out = color8[0]

    return (select_out, node_index, color_out)


# speedup vs baseline: 1.0081x; 1.0081x over previous
import functools

import jax
import jax.numpy as jnp
from jax import lax
from jax.experimental import pallas as pl
from jax.experimental.pallas import tpu as pltpu
from jax.experimental.pallas import tpu_sc as plsc

N = 10000
H = 128
ANN = 2
E = 320000
ELIG = 5000
A_SIZE = 32
FC = 64

NC = 2
NS = 16
L = 16
NW = NC * NS

CHUNK = 128
CHUNKS_PER_TILE = 79
TOTAL_CHUNKS_PAD = NW * CHUNKS_PER_TILE
E_PAD = TOTAL_CHUNKS_PAD * CHUNK
ACC_ROWS = 10240
STRIPE = ACC_ROWS // NS
LAST_ROWS = N - (NS - 1) * STRIPE

BN = 5000


def _stage_a_body(h0_ref, annp_ref, wmT_ref, bm_ref, whT_ref, bh_ref,
                  h_ref, hm_ref, gh_ref):
    h = h0_ref[...] + annp_ref[...]
    h_ref[...] = h
    hm_ref[...] = jnp.dot(h, wmT_ref[...],
                          preferred_element_type=jnp.float32) + bm_ref[...]
    gh_ref[...] = jnp.dot(h, whT_ref[...],
                          preferred_element_type=jnp.float32) + bh_ref[...]


def _stage_a(h0, annp, wmT, bm, whT, bh):
    grid = (N // BN,)
    row_spec = pl.BlockSpec((BN, H), lambda i: (i, 0))
    full = lambda shape: pl.BlockSpec(shape, lambda i: (0,) * len(shape))
    return pl.pallas_call(
        _stage_a_body,
        grid=grid,
        in_specs=[row_spec, row_spec, full((H, H)), full((1, H)),
                  full((H, 3 * H)), full((1, 3 * H))],
        out_specs=[row_spec, row_spec,
                   pl.BlockSpec((BN, 3 * H), lambda i: (i, 0))],
        out_shape=[jax.ShapeDtypeStruct((N, H), jnp.float32),
                   jax.ShapeDtypeStruct((N, H), jnp.float32),
                   jax.ShapeDtypeStruct((N, 3 * H), jnp.float32)],
    )(h0, annp, wmT, bm, whT, bh)


@functools.cache
def _mesh():
    return plsc.VectorSubcoreMesh(core_axis_name="c", subcore_axis_name="s",
                                  num_cores=NC, num_subcores=NS)


@functools.cache
def _build_scatter_kernel():
  @functools.partial(
      pl.kernel,
      out_type=jax.ShapeDtypeStruct((NC, N, H), jnp.float32),
      mesh=_mesh(),
      scratch_types=[
          pltpu.VMEM((CHUNKS_PER_TILE, CHUNK), jnp.int32),
          pltpu.VMEM((CHUNKS_PER_TILE, CHUNK), jnp.int32),
          pltpu.VMEM((CHUNK, H), jnp.float32),
          pltpu.VMEM_SHARED((ACC_ROWS, H), jnp.float32),
          pltpu.SemaphoreType.DMA,
      ],
      compiler_params=pltpu.CompilerParams(needs_layout_passes=False),
  )
  def _scatter_kernel(hm_hbm, src_hbm, dst_hbm, zeros_hbm, out_hbm,
                      sidx_v, didx_v, rows_v, acc_sh, sem):
    c = lax.axis_index("c")
    s = lax.axis_index("s")
    wid = s * NC + c

    r0 = s * STRIPE
    pltpu.sync_copy(zeros_hbm, acc_sh.at[pl.ds(r0, STRIPE)])
    plsc.subcore_barrier()

    pltpu.sync_copy(src_hbm.at[wid], sidx_v)
    pltpu.sync_copy(dst_hbm.at[wid], didx_v)

    def body(t, carry):
        pltpu.async_copy(hm_hbm.at[sidx_v.at[t]], rows_v, sem).wait()
        pltpu.sync_copy(rows_v, acc_sh.at[didx_v.at[t]], add=True)
        return carry

    lax.fori_loop(0, CHUNKS_PER_TILE, body, 0)

    plsc.subcore_barrier()
    @pl.when(s < NS - 1)
    def _():
        pltpu.sync_copy(acc_sh.at[pl.ds(r0, STRIPE)],
                        out_hbm.at[c, pl.ds(r0, STRIPE)])

    @pl.when(s == NS - 1)
    def _():
        pltpu.sync_copy(acc_sh.at[pl.ds((NS - 1) * STRIPE, LAST_ROWS)],
                        out_hbm.at[c, pl.ds((NS - 1) * STRIPE, LAST_ROWS)])

  return _scatter_kernel


def _stage_c_body(p0_ref, p1_ref, h_ref, gh_ref, wiT_ref, bi_ref,
                  w1T_ref, b1_ref, w2T_ref, b2_ref, w3_ref, b3_ref,
                  hid_ref, sel_ref):
    incoming = p0_ref[...] + p1_ref[...]
    gi = jnp.dot(incoming, wiT_ref[...],
                 preferred_element_type=jnp.float32) + bi_ref[...]
    gh = gh_ref[...]
    h = h_ref[...]
    r = jax.nn.sigmoid(gi[:, :H] + gh[:, :H])
    z = jax.nn.sigmoid(gi[:, H:2 * H] + gh[:, H:2 * H])
    n = jnp.tanh(gi[:, 2 * H:] + r * gh[:, 2 * H:])
    hid = (1.0 - z) * n + z * h
    hid_ref[...] = hid
    x = jax.nn.relu(jnp.dot(hid, w1T_ref[...],
                            preferred_element_type=jnp.float32) + b1_ref[...])
    x = jax.nn.relu(jnp.dot(x, w2T_ref[...],
                            preferred_element_type=jnp.float32) + b2_ref[...])
    sel_ref[...] = (jnp.sum(x * w3_ref[...], axis=1, keepdims=True)
                    + b3_ref[...])


def _stage_c(p0, p1, h, gh, wiT, bi, w1T, b1, w2T, b2, w3, b3):
    grid = (N // BN,)
    row_spec = pl.BlockSpec((BN, H), lambda i: (i, 0))
    row3_spec = pl.BlockSpec((BN, 3 * H), lambda i: (i, 0))
    full = lambda shape: pl.BlockSpec(shape, lambda i: (0,) * len(shape))
    return pl.pallas_call(
        _stage_c_body,
        grid=grid,
        in_specs=[row_spec, row_spec, row_spec, row3_spec,
                  full((H, 3 * H)), full((1, 3 * H)),
                  full((H, FC)), full((1, FC)),
                  full((FC, FC)), full((1, FC)),
                  full((1, FC)), full((1, 1))],
        out_specs=[row_spec, pl.BlockSpec((BN, 1), lambda i: (i, 0))],
        out_shape=[jax.ShapeDtypeStruct((N, H), jnp.float32),
                   jax.ShapeDtypeStruct((N, 1), jnp.float32)],
    )(p0, p1, h, gh, wiT, bi, w1T, b1, w2T, b2, w3, b3)


_N_FULL = ELIG // L
_REM = ELIG - _N_FULL * L
_EC = 128
_E_CHUNKS = -(-ELIG // _EC)
_ELIG_PAD = _E_CHUNKS * _EC
_BIG = 2147483647


@functools.cache
def _build_argmax_kernel():
  @functools.partial(
      pl.kernel,
      out_type=(jax.ShapeDtypeStruct((L,), jnp.int32),
                jax.ShapeDtypeStruct((L, H), jnp.float32)),
      mesh=_mesh(),
      scratch_types=[
          pltpu.VMEM((_E_CHUNKS, _EC), jnp.int32),
          pltpu.VMEM((_E_CHUNKS, _EC), jnp.float32),
          pltpu.VMEM((L,), jnp.int32),
          pltpu.VMEM((L, H), jnp.float32),
          pltpu.SemaphoreType.DMA,
      ],
      compiler_params=pltpu.CompilerParams(needs_layout_passes=False),
  )
  def _argmax_kernel(sel_hbm, elig_hbm, hid_hbm, idx_out, row_out,
                     elig_v, vals_v, nidx_v, row_v, sem):
    c = lax.axis_index("c")
    s = lax.axis_index("s")

    @pl.when(jnp.logical_and(c == 0, s == 0))
    def _():
        pltpu.sync_copy(elig_hbm, elig_v)
        descs = [
            pltpu.async_copy(sel_hbm.at[elig_v.at[t]], vals_v.at[t], sem)
            for t in range(_E_CHUNKS)
        ]
        for d in descs:
            d.wait()

        neg = jnp.full((L,), -3.0e38, jnp.float32)
        lane = lax.iota(jnp.int32, L)
        per_row = _EC // L

        def body(j, carry):
            best_val, best_pos, best_node = carry
            row = j // per_row
            col = (j % per_row) * L
            vals = vals_v[row, pl.ds(col, L)]
            nodes = elig_v[row, pl.ds(col, L)]
            pos = j * L + lane
            better = vals > best_val
            return (jnp.where(better, vals, best_val),
                    jnp.where(better, pos, best_pos),
                    jnp.where(better, nodes, best_node))

        bv, bp, bn = lax.fori_loop(
            0, _N_FULL, body,
            (neg, jnp.zeros((L,), jnp.int32), jnp.zeros((L,), jnp.int32)))
        tmask = lane < _REM
        row = _N_FULL // per_row
        col = (_N_FULL % per_row) * L
        vals_t = jnp.where(tmask, vals_v[row, pl.ds(col, L)], neg)
        nodes_t = elig_v[row, pl.ds(col, L)]
        pos_t = _N_FULL * L + lane
        better = vals_t > bv
        bv = jnp.where(better, vals_t, bv)
        bp = jnp.where(better, pos_t, bp)
        bn = jnp.where(better, nodes_t, bn)
        m = jnp.max(bv)
        rel = jnp.min(jnp.where(bv == m, bp, _BIG))
        node = jnp.min(jnp.where(bp == rel, bn, _BIG))
        nidx_v[...] = jnp.full((L,), node, jnp.int32)
        pltpu.sync_copy(nidx_v, idx_out)
        pltpu.async_copy(hid_hbm.at[nidx_v], row_v, sem).wait()
        pltpu.sync_copy(row_v, row_out)

  return _argmax_kernel


def _stage_e_body(row_ref, w1T_ref, b1_ref, w2T_ref, b2_ref, w3T_ref, b3_ref,
                  out_ref):
    x = row_ref[...]
    x = jax.nn.relu(jnp.dot(x, w1T_ref[...],
                            preferred_element_type=jnp.float32) + b1_ref[...])
    x = jax.nn.relu(jnp.dot(x, w2T_ref[...],
                            preferred_element_type=jnp.float32) + b2_ref[...])
    out_ref[...] = jnp.dot(x, w3T_ref[...],
                           preferred_element_type=jnp.float32) + b3_ref[...]


def _stage_e(row8, w1T, b1, w2T, b2, w3T, b3):
    return pl.pallas_call(
        _stage_e_body,
        out_shape=jax.ShapeDtypeStruct((8, A_SIZE), jnp.float32),
    )(row8, w1T, b1, w2T, b2, w3T, b3)


def kernel(initial_node_representation, annotations, edge_index, eligibleNodes,
           W_msg, b_msg, gru_wi, gru_wh, gru_bi, gru_bh,
           sel_w1, sel_b1, sel_w2, sel_b2, sel_w3, sel_b3,
           col_w1, col_b1, col_w2, col_b2, col_w3, col_b3):
    annp = jnp.pad(annotations, ((0, 0), (0, H - ANN)))

    h, hm, gh = _stage_a(
        initial_node_representation, annp,
        W_msg.T, b_msg.reshape(1, H),
        gru_wh.T, gru_bh.reshape(1, 3 * H))

    pad = E_PAD - E
    src_p = jnp.concatenate(
        [edge_index[0], jnp.zeros((pad,), jnp.int32)]
    ).reshape(NW, CHUNKS_PER_TILE, CHUNK)
    dst_p = jnp.concatenate(
        [edge_index[1], jnp.full((pad,), N, jnp.int32)]
    ).reshape(NW, CHUNKS_PER_TILE, CHUNK)
    zeros = jnp.zeros((STRIPE, H), jnp.float32)

    partials = _build_scatter_kernel()(hm, src_p, dst_p, zeros)

    hidden, select_out = _stage_c(
        partials[0], partials[1], h, gh,
        gru_wi.T, gru_bi.reshape(1, 3 * H),
        sel_w1.T, sel_b1.reshape(1, FC),
        sel_w2.T, sel_b2.reshape(1, FC),
        sel_w3.reshape(1, FC), sel_b3.reshape(1, 1))

    elig_p = jnp.concatenate(
        [eligibleNodes, jnp.zeros((_ELIG_PAD - ELIG,), jnp.int32)]
    ).reshape(_E_CHUNKS, _EC)
    idx_vec, row = _build_argmax_kernel()(select_out.reshape(N),
                                          elig_p, hidden)
    node_index = idx_vec[0]

    color8 = _stage_e(
        row[:8],
        col_w1.T, col_b1.reshape(1, FC),
        col_w2.T, col_b2.reshape(1, FC),
        col_w3.T, col_b3.reshape(1, A_SIZE))
    color_out = color8[0]

    return (select_out, node_index, color_out)
